# baseline (speedup 1.0000x reference)
import functools

import jax
import jax.numpy as jnp
from jax import lax
from jax.experimental import pallas as pl
from jax.experimental.pallas import tpu as pltpu
from jax.experimental.pallas import tpu_sc as plsc

N = 10000
E = 320000
D = 128
HB = 64
NC, NS = 2, 16
NW = NC * NS
CHUNK = 128
NCHUNKS = 2592
BASE_CH = NCHUNKS // NW
E_PAD = NCHUNKS * CHUNK
NP = 10112
RPS = NP // NS

_mesh = plsc.VectorSubcoreMesh(
    core_axis_name="c", subcore_axis_name="s", num_cores=NC, num_subcores=NS)


NBUF = 3
GROUPS = BASE_CH // NBUF


def _sc_agg_body(with_cnt, *refs):
    if with_cnt:
        (x_hbm, edge_hbm, zrow_hbm, ones_hbm,
         agg_hbm, cnt_hbm, acc_sh, ids_v, rows_v, gsem, ssem) = refs
    else:
        (x_hbm, edge_hbm, zrow_hbm,
         agg_hbm, acc_sh, ids_v, rows_v, gsem, ssem) = refs

    cid = lax.axis_index("c")
    sid = lax.axis_index("s")
    wid = sid * NC + cid
    row0 = sid * RPS
    c0 = wid * BASE_CH

    pltpu.sync_copy(zrow_hbm, acc_sh.at[pl.ds(row0, RPS), :])
    plsc.subcore_barrier()

    def fire_gather(b):
        return pltpu.async_copy(x_hbm.at[ids_v.at[b, 0]], rows_v.at[b],
                                gsem.at[b])

    def fire_scatter(b):
        pltpu.async_copy(rows_v.at[b], acc_sh.at[ids_v.at[b, 1]],
                         ssem.at[b], add=True)

    def wait_scatter(b):
        pltpu.make_async_copy(x_hbm.at[pl.ds(0, CHUNK), :], rows_v.at[b],
                              ssem.at[b]).wait()

    @pl.loop(0, GROUPS)
    def _(g):
        for b in range(NBUF):
            @pl.when(g > 0)
            def _():
                wait_scatter(b)
        pltpu.sync_copy(edge_hbm.at[pl.ds(c0 + g * NBUF, NBUF)], ids_v)
        descs = [fire_gather(b) for b in range(NBUF)]
        for b in range(NBUF):
            descs[b].wait()
            fire_scatter(b)

    for b in range(NBUF):
        wait_scatter(b)

    plsc.subcore_barrier()
    pltpu.sync_copy(acc_sh.at[pl.ds(row0, RPS), :],
                    agg_hbm.at[cid, pl.ds(row0, RPS), :])

    if not with_cnt:
        return

    pltpu.sync_copy(zrow_hbm, acc_sh.at[pl.ds(row0, RPS), :])
    pltpu.sync_copy(ones_hbm, rows_v.at[0])
    plsc.subcore_barrier()

    def fire_cnt(b):
        pltpu.async_copy(rows_v.at[0], acc_sh.at[ids_v.at[b, 1]],
                         ssem.at[b], add=True)

    @pl.loop(0, GROUPS)
    def _(g):
        for b in range(NBUF):
            @pl.when(g > 0)
            def _():
                wait_scatter(b)
        pltpu.sync_copy(edge_hbm.at[pl.ds(c0 + g * NBUF, NBUF)], ids_v)
        for b in range(NBUF):
            fire_cnt(b)

    for b in range(NBUF):
        wait_scatter(b)

    plsc.subcore_barrier()
    pltpu.sync_copy(acc_sh.at[pl.ds(row0, RPS), :],
                    cnt_hbm.at[cid, pl.ds(row0, RPS), :])


def _sc_agg(x, edge3d, with_cnt):
    zrow = jnp.zeros((RPS, D), jnp.float32)
    scratch = [pltpu.VMEM_SHARED((NP, D), jnp.float32),
               pltpu.VMEM((NBUF, 2, CHUNK), jnp.int32),
               pltpu.VMEM((NBUF, CHUNK, D), jnp.float32),
               pltpu.SemaphoreType.DMA((NBUF,)),
               pltpu.SemaphoreType.DMA((NBUF,))]
    if with_cnt:
        out_type = [jax.ShapeDtypeStruct((NC, NP, D), jnp.float32),
                    jax.ShapeDtypeStruct((NC, NP, D), jnp.float32)]
        ones = jnp.ones((CHUNK, D), jnp.float32)
        k = pl.kernel(functools.partial(_sc_agg_body, True),
                      out_type=out_type, mesh=_mesh, scratch_types=scratch)
        return k(x, edge3d, zrow, ones)
    out_type = [jax.ShapeDtypeStruct((NC, NP, D), jnp.float32)]
    k = pl.kernel(functools.partial(_sc_agg_body, False),
                  out_type=out_type, mesh=_mesh, scratch_types=scratch)
    return k(x, edge3d, zrow)[0]


def _tc_pre(x, W):
    def body(x_ref, w_ref, o_ref):
        o_ref[...] = jnp.dot(x_ref[...], w_ref[...],
                             preferred_element_type=jnp.float32)
    return pl.pallas_call(
        body, out_shape=jax.ShapeDtypeStruct((N, D), jnp.float32))(x, W)


def _tc_post(agg, cnt, hr, Wl, brow):
    def body(a_ref, c_ref, hr_ref, w_ref, b_ref, o_ref):
        s = a_ref[0, 0:N, :] + a_ref[1, 0:N, :]
        c = c_ref[0, 0:N, 0:1] + c_ref[1, 0:N, 0:1]
        mean = s * (1.0 / jnp.maximum(c, 1.0))
        o_ref[...] = (jnp.dot(mean, w_ref[...],
                              preferred_element_type=jnp.float32)
                      + b_ref[...] + hr_ref[...])
    return pl.pallas_call(
        body, out_shape=jax.ShapeDtypeStruct((N, D), jnp.float32))(
            agg, cnt, hr, Wl, brow)


def _tc_vn_in2(agg, cnt, hr, Wl, brow, hbcol, hbrow, ve,
               Wm0a, bm0a, Wm0b, bm0b, Wm1a, bm1a, Wm1b, bm1b):
    def body(a_ref, c_ref, hr_ref, w_ref, b_ref, hc_ref, hr2_ref, ve_ref,
             w0a, b0a, w0b, b0b, w1a, b1a, w1b, b1b, o_ref):
        s = a_ref[0, 0:N, :] + a_ref[1, 0:N, :]
        c = c_ref[0, 0:N, 0:1] + c_ref[1, 0:N, 0:1]
        mean = s * (1.0 / jnp.maximum(c, 1.0))
        out1 = (jnp.dot(mean, w_ref[...], preferred_element_type=jnp.float32)
                + b_ref[...] + hr_ref[...])
        ohT = (lax.broadcasted_iota(jnp.int32, (HB, N), 0)
               == hr2_ref[...]).astype(jnp.float32)
        pool = jnp.dot(ohT, out1, preferred_element_type=jnp.float32)
        vn = pool + 2.0 * ve_ref[...]
        vn = jnp.maximum(jnp.dot(vn, w0a[...],
                                 preferred_element_type=jnp.float32)
                         + b0a[...], 0.0)
        vn = jnp.maximum(jnp.dot(vn, w0b[...],
                                 preferred_element_type=jnp.float32)
                         + b0b[...], 0.0)
        vn = jnp.maximum(jnp.dot(vn, w1a[...],
                                 preferred_element_type=jnp.float32)
                         + b1a[...], 0.0)
        vn = jnp.maximum(jnp.dot(vn, w1b[...],
                                 preferred_element_type=jnp.float32)
                         + b1b[...], 0.0)
        oh = (lax.broadcasted_iota(jnp.int32, (N, HB), 1)
              == hc_ref[...]).astype(jnp.float32)
        o_ref[...] = out1 + jnp.dot(oh, vn,
                                    preferred_element_type=jnp.float32)
    args = (agg, cnt, hr, Wl, brow, hbcol, hbrow, ve,
            Wm0a, bm0a[None, :], Wm0b, bm0b[None, :],
            Wm1a, bm1a[None, :], Wm1b, bm1b[None, :])
    return pl.pallas_call(
        body, out_shape=jax.ShapeDtypeStruct((N, D), jnp.float32))(*args)


def kernel(x, edge_index, h_blocks, h_levels, h_num, vn_emb,
           Wl0, bl0, Wr0, Wl1, bl1, Wr1, Wl2, bl2, Wr2,
           Wm0a, bm0a, Wm0b, bm0b, Wm1a, bm1a, Wm1b, bm1b):
    npad = E_PAD - E
    pad_src = (jnp.arange(npad, dtype=jnp.int32) * 37) % N
    pad_dst = N + (jnp.arange(npad, dtype=jnp.int32) % (NP - N))
    src2d = jnp.concatenate([edge_index[0], pad_src]).reshape(NCHUNKS, 1, CHUNK)
    dst2d = jnp.concatenate([edge_index[1], pad_dst]).reshape(NCHUNKS, 1, CHUNK)
    edge3d = jnp.concatenate([src2d, dst2d], axis=1)
    hbcol = h_blocks.reshape(N, 1)
    hbrow = h_blocks.reshape(1, N)

    agg0, cnt = _sc_agg(x, edge3d, with_cnt=True)
    hr0 = _tc_pre(x, Wr0)
    in1 = _tc_post(agg0, cnt, hr0, Wl0, (bl0 + vn_emb[0])[None, :])

    agg1 = _sc_agg(in1, edge3d, with_cnt=False)
    hr1 = _tc_pre(in1, Wr1)
    in2 = _tc_vn_in2(agg1, cnt, hr1, Wl1, bl1[None, :], hbcol, hbrow, vn_emb,
                     Wm0a, bm0a, Wm0b, bm0b, Wm1a, bm1a, Wm1b, bm1b)

    agg2 = _sc_agg(in2, edge3d, with_cnt=False)
    hr2 = _tc_pre(in2, Wr2)
    out2 = _tc_post(agg2, cnt, hr2, Wl2, bl2[None, :])
    return out2


# speedup vs baseline: 1.0846x; 1.0846x over previous
import functools

import jax
import jax.numpy as jnp
from jax import lax
from jax.experimental import pallas as pl
from jax.experimental.pallas import tpu as pltpu
from jax.experimental.pallas import tpu_sc as plsc

N = 10000
E = 320000
D = 128
HB = 64
NC, NS = 2, 16
NW = NC * NS
CHUNK = 128
NCHUNKS = 2560
BASE_CH = NCHUNKS // NW
E_PAD = NCHUNKS * CHUNK
NP = 10240
RPS = NP // NS

_mesh = plsc.VectorSubcoreMesh(
    core_axis_name="c", subcore_axis_name="s", num_cores=NC, num_subcores=NS)


NBUF = 2
HALF = BASE_CH // 2
GR_H = HALF // NBUF


def _sc_agg_body(with_cnt, *refs):
    if with_cnt:
        (x_hbm, src_hbm, dst_hbm, zrow_hbm, ones_hbm,
         agg_hbm, cnt_hbm, acc_sh, sids_v, dids_v, rows_v, gsem, ssem) = refs
    else:
        (x_hbm, src_hbm, dst_hbm, zrow_hbm,
         agg_hbm, acc_sh, sids_v, dids_v, rows_v, gsem, ssem) = refs

    cid = lax.axis_index("c")
    sid = lax.axis_index("s")
    wid = sid * NC + cid
    row0 = sid * RPS
    c0 = wid * BASE_CH

    pltpu.sync_copy(zrow_hbm, acc_sh.at[pl.ds(row0, RPS), :])
    plsc.subcore_barrier()

    def fire_gather(b, c):
        return pltpu.async_copy(x_hbm.at[sids_v.at[c]], rows_v.at[b],
                                gsem.at[b])

    def fire_scatter(b, c):
        pltpu.async_copy(rows_v.at[b], acc_sh.at[dids_v.at[c]],
                         ssem.at[b], add=True)

    def wait_scatter(b):
        pltpu.make_async_copy(x_hbm.at[pl.ds(0, CHUNK), :], rows_v.at[b],
                              ssem.at[b]).wait()

    for h in range(2):
        pltpu.sync_copy(src_hbm.at[pl.ds(c0 + h * HALF, HALF), :], sids_v)
        pltpu.sync_copy(dst_hbm.at[pl.ds(c0 + h * HALF, HALF), :], dids_v)

        @pl.loop(0, GR_H)
        def _(g):
            descs = []
            for b in range(NBUF):
                @pl.when(g > 0)
                def _():
                    wait_scatter(b)
                descs.append(fire_gather(b, g * NBUF + b))
            for b in range(NBUF):
                descs[b].wait()
                fire_scatter(b, g * NBUF + b)

        for b in range(NBUF):
            wait_scatter(b)

    plsc.subcore_barrier()
    pltpu.sync_copy(acc_sh.at[pl.ds(row0, RPS), :],
                    agg_hbm.at[cid, pl.ds(row0, RPS), :])

    if not with_cnt:
        return

    pltpu.sync_copy(zrow_hbm, acc_sh.at[pl.ds(row0, RPS), :])
    pltpu.sync_copy(ones_hbm, rows_v.at[0])
    plsc.subcore_barrier()

    def fire_cnt(b, c):
        pltpu.async_copy(rows_v.at[0], acc_sh.at[dids_v.at[c]],
                         ssem.at[b], add=True)

    for h in range(2):
        pltpu.sync_copy(dst_hbm.at[pl.ds(c0 + h * HALF, HALF), :], dids_v)

        @pl.loop(0, GR_H)
        def _(g):
            for b in range(NBUF):
                @pl.when(g > 0)
                def _():
                    wait_scatter(b)
                fire_cnt(b, g * NBUF + b)

        for b in range(NBUF):
            wait_scatter(b)

    plsc.subcore_barrier()
    pltpu.sync_copy(acc_sh.at[pl.ds(row0, RPS), :],
                    cnt_hbm.at[cid, pl.ds(row0, RPS), :])


def _sc_agg(x, src2d, dst2d, with_cnt):
    zrow = jnp.zeros((RPS, D), jnp.float32)
    scratch = [pltpu.VMEM_SHARED((NP, D), jnp.float32),
               pltpu.VMEM((HALF, CHUNK), jnp.int32),
               pltpu.VMEM((HALF, CHUNK), jnp.int32),
               pltpu.VMEM((NBUF, CHUNK, D), jnp.float32),
               pltpu.SemaphoreType.DMA((NBUF,)),
               pltpu.SemaphoreType.DMA((NBUF,))]
    if with_cnt:
        out_type = [jax.ShapeDtypeStruct((NC, NP, D), jnp.float32),
                    jax.ShapeDtypeStruct((NC, NP, D), jnp.float32)]
        ones = jnp.ones((CHUNK, D), jnp.float32)
        k = pl.kernel(functools.partial(_sc_agg_body, True),
                      out_type=out_type, mesh=_mesh, scratch_types=scratch)
        return k(x, src2d, dst2d, zrow, ones)
    out_type = [jax.ShapeDtypeStruct((NC, NP, D), jnp.float32)]
    k = pl.kernel(functools.partial(_sc_agg_body, False),
                  out_type=out_type, mesh=_mesh, scratch_types=scratch)
    return k(x, src2d, dst2d, zrow)[0]


def _tc_pre(x, W):
    def body(x_ref, w_ref, o_ref):
        o_ref[...] = jnp.dot(x_ref[...], w_ref[...],
                             preferred_element_type=jnp.float32)
    return pl.pallas_call(
        body, out_shape=jax.ShapeDtypeStruct((N, D), jnp.float32))(x, W)


def _tc_post(agg, cnt, hr, Wl, brow):
    def body(a_ref, c_ref, hr_ref, w_ref, b_ref, o_ref):
        s = a_ref[0, 0:N, :] + a_ref[1, 0:N, :]
        c = c_ref[0, 0:N, 0:1] + c_ref[1, 0:N, 0:1]
        mean = s * (1.0 / jnp.maximum(c, 1.0))
        o_ref[...] = (jnp.dot(mean, w_ref[...],
                              preferred_element_type=jnp.float32)
                      + b_ref[...] + hr_ref[...])
    return pl.pallas_call(
        body, out_shape=jax.ShapeDtypeStruct((N, D), jnp.float32))(
            agg, cnt, hr, Wl, brow)


def _tc_vn_in2(agg, cnt, hr, Wl, brow, hbcol, hbrow, ve,
               Wm0a, bm0a, Wm0b, bm0b, Wm1a, bm1a, Wm1b, bm1b):
    def body(a_ref, c_ref, hr_ref, w_ref, b_ref, hc_ref, hr2_ref, ve_ref,
             w0a, b0a, w0b, b0b, w1a, b1a, w1b, b1b, o_ref):
        s = a_ref[0, 0:N, :] + a_ref[1, 0:N, :]
        c = c_ref[0, 0:N, 0:1] + c_ref[1, 0:N, 0:1]
        mean = s * (1.0 / jnp.maximum(c, 1.0))
        out1 = (jnp.dot(mean, w_ref[...], preferred_element_type=jnp.float32)
                + b_ref[...] + hr_ref[...])
        ohT = (lax.broadcasted_iota(jnp.int32, (HB, N), 0)
               == hr2_ref[...]).astype(jnp.float32)
        pool = jnp.dot(ohT, out1, preferred_element_type=jnp.float32)
        vn = pool + 2.0 * ve_ref[...]
        vn = jnp.maximum(jnp.dot(vn, w0a[...],
                                 preferred_element_type=jnp.float32)
                         + b0a[...], 0.0)
        vn = jnp.maximum(jnp.dot(vn, w0b[...],
                                 preferred_element_type=jnp.float32)
                         + b0b[...], 0.0)
        vn = jnp.maximum(jnp.dot(vn, w1a[...],
                                 preferred_element_type=jnp.float32)
                         + b1a[...], 0.0)
        vn = jnp.maximum(jnp.dot(vn, w1b[...],
                                 preferred_element_type=jnp.float32)
                         + b1b[...], 0.0)
        oh = (lax.broadcasted_iota(jnp.int32, (N, HB), 1)
              == hc_ref[...]).astype(jnp.float32)
        o_ref[...] = out1 + jnp.dot(oh, vn,
                                    preferred_element_type=jnp.float32)
    args = (agg, cnt, hr, Wl, brow, hbcol, hbrow, ve,
            Wm0a, bm0a[None, :], Wm0b, bm0b[None, :],
            Wm1a, bm1a[None, :], Wm1b, bm1b[None, :])
    return pl.pallas_call(
        body, out_shape=jax.ShapeDtypeStruct((N, D), jnp.float32))(*args)


def kernel(x, edge_index, h_blocks, h_levels, h_num, vn_emb,
           Wl0, bl0, Wr0, Wl1, bl1, Wr1, Wl2, bl2, Wr2,
           Wm0a, bm0a, Wm0b, bm0b, Wm1a, bm1a, Wm1b, bm1b):
    npad = E_PAD - E
    pad_src = (jnp.arange(npad, dtype=jnp.int32) * 37) % N
    pad_dst = N + (jnp.arange(npad, dtype=jnp.int32) % (NP - N))
    src2d = jnp.concatenate([edge_index[0], pad_src]).reshape(NCHUNKS, CHUNK)
    dst2d = jnp.concatenate([edge_index[1], pad_dst]).reshape(NCHUNKS, CHUNK)
    hbcol = h_blocks.reshape(N, 1)
    hbrow = h_blocks.reshape(1, N)

    agg0, cnt = _sc_agg(x, src2d, dst2d, with_cnt=True)
    hr0 = _tc_pre(x, Wr0)
    in1 = _tc_post(agg0, cnt, hr0, Wl0, (bl0 + vn_emb[0])[None, :])

    agg1 = _sc_agg(in1, src2d, dst2d, with_cnt=False)
    hr1 = _tc_pre(in1, Wr1)
    in2 = _tc_vn_in2(agg1, cnt, hr1, Wl1, bl1[None, :], hbcol, hbrow, vn_emb,
                     Wm0a, bm0a, Wm0b, bm0b, Wm1a, bm1a, Wm1b, bm1b)

    agg2 = _sc_agg(in2, src2d, dst2d, with_cnt=False)
    hr2 = _tc_pre(in2, Wr2)
    out2 = _tc_post(agg2, cnt, hr2, Wl2, bl2[None, :])
    return out2
